# baseline (speedup 1.0000x reference)
import functools

import jax
import jax.numpy as jnp
from jax import lax
from jax.experimental import pallas as pl
from jax.experimental.pallas import tpu as pltpu
from jax.experimental.pallas import tpu_sc as plsc

B, NS, NR, F, K = 4, 2048, 2048, 128, 16
NSTOT = B * NS
NE = NSTOT * K

BS = 256

NW = 32
SPW = NSTOT // NW
CS = 4
CHUNKS = SPW // CS


def _topk_body(s_ref, r_ref, adj_ref, idx_ref):
    b = pl.program_id(0)
    s = s_ref[0]
    r = r_ref[0]
    s2 = jnp.sum(s * s, axis=1, keepdims=True)
    r2 = jnp.sum(r * r, axis=1)[None, :]
    mixed = lax.dot_general(s, r, (((1,), (1,)), ((), ())),
                            preferred_element_type=jnp.float32)
    d = jnp.abs(s2 + r2 - 2.0 * mixed)
    col = lax.broadcasted_iota(jnp.int32, (BS, NR), 1)
    kcol = lax.broadcasted_iota(jnp.int32, (BS, K), 1)
    offset = NSTOT + b * NR
    idxs = jnp.zeros((BS, K), jnp.int32)
    inf = jnp.float32(jnp.inf)
    colw = lax.broadcasted_iota(jnp.int32, (NR, 3), 0)
    sw = lax.broadcasted_iota(jnp.int32, (NR, 3), 1)
    w = jnp.where(sw == 0, (colw >> 4).astype(jnp.float32),
                  jnp.where(sw == 1, (colw & 15).astype(jnp.float32), 1.0))
    for k in range(K):
        m = jnp.min(d, axis=1, keepdims=True)
        eqf = jnp.where(d <= m, 1.0, 0.0).astype(jnp.float32)
        acc = lax.dot_general(eqf, w, (((1,), (0,)), ((), ())),
                              preferred_element_type=jnp.float32)
        s_hi, s_lo, cnt = acc[:, 0:1], acc[:, 1:2], acc[:, 2:3]
        sel_fast = (16.0 * s_hi + s_lo).astype(jnp.int32)
        bad = jnp.any(cnt > 1.5)
        sel = lax.cond(
            bad,
            lambda: jnp.min(jnp.where(d <= m, col, NR), axis=1, keepdims=True),
            lambda: sel_fast,
        )
        d = jnp.where(col == sel, inf, d)
        idxs = jnp.where(kcol == k, sel + offset, idxs)
    adj_ref[0] = (d == inf).astype(jnp.float32)
    idx_ref[0] = idxs


def _topk_call(senders, receivers):
    return pl.pallas_call(
        _topk_body,
        grid=(B, NS // BS),
        in_specs=[
            pl.BlockSpec((1, BS, F), lambda b, i: (b, i, 0)),
            pl.BlockSpec((1, NR, F), lambda b, i: (b, 0, 0)),
        ],
        out_specs=[
            pl.BlockSpec((1, BS, NR), lambda b, i: (b, i, 0)),
            pl.BlockSpec((1, BS, K), lambda b, i: (b, i, 0)),
        ],
        out_shape=[
            jax.ShapeDtypeStruct((B, NS, NR), jnp.float32),
            jax.ShapeDtypeStruct((B, NS, K), jnp.int32),
        ],
    )(senders, receivers)


def _sc_gather_body(table_hbm, idxg_hbm, out_hbm, idx_all,
                    gidx0, gidx1, gv0, gv1, gs0, gs1, ws0, ws1):
    wid = lax.axis_index("s") * 2 + lax.axis_index("c")
    sender_base = wid * SPW
    pltpu.sync_copy(idxg_hbm.at[pl.ds(sender_base * K, SPW * K)], idx_all)
    lane = lax.iota(jnp.int32, 16)
    gidx = (gidx0, gidx1)
    gv = (gv0, gv1)
    gs = (gs0, gs1)
    ws = (ws0, ws1)
    HR = 2 * CS * K

    def fill_and_fire(c, b):
        for i in range(CS):
            off = c * (CS * K) + i * K
            v = idx_all[pl.ds(off, K)]
            sk, _ = plsc.sort_key_val(v, v)
            sg = sender_base + c * CS + i
            splat = jnp.broadcast_to(sg, (16,)).astype(jnp.int32)
            pos_e = 2 * lane + (2 * K) * i
            plsc.store_scatter(gidx[b], [pos_e], splat)
            plsc.store_scatter(gidx[b], [pos_e + 1], sk)
        pltpu.make_async_copy(table_hbm.at[gidx[b]], gv[b], gs[b]).start()

    def out_slice(c):
        return out_hbm.at[pl.ds(sender_base * (2 * K) + c * HR, HR)]

    def drain_write(c, b):
        pltpu.make_async_copy(gv[b], out_slice(c), ws[b]).wait()

    def write_start(c, b):
        pltpu.make_async_copy(gv[b], out_slice(c), ws[b]).start()

    def wait_gather(b):
        pltpu.make_async_copy(table_hbm.at[gidx[b]], gv[b], gs[b]).wait()

    fill_and_fire(0, 0)

    def body(g, carry):
        c1 = 2 * g + 1

        @pl.when(g > 0)
        def _():
            drain_write(c1 - 2, 1)
        fill_and_fire(c1, 1)
        wait_gather(0)
        write_start(2 * g, 0)
        c2 = 2 * g + 2

        @pl.when(c2 < CHUNKS)
        def _():
            drain_write(c2 - 2, 0)
            fill_and_fire(c2, 0)
        wait_gather(1)
        write_start(c1, 1)
        return carry

    lax.fori_loop(0, CHUNKS // 2, body, 0)
    drain_write(CHUNKS - 2, 0)
    drain_write(CHUNKS - 1, 1)


@functools.cache
def _sc_gather_fn():
    mesh = plsc.VectorSubcoreMesh(core_axis_name="c", subcore_axis_name="s")
    return pl.kernel(
        _sc_gather_body,
        mesh=mesh,
        compiler_params=pltpu.CompilerParams(needs_layout_passes=False),
        out_type=jax.ShapeDtypeStruct((2 * NE, F), jnp.float32),
        scratch_types=[
            pltpu.VMEM((SPW * K,), jnp.int32),
            pltpu.VMEM((2 * CS * K,), jnp.int32),
            pltpu.VMEM((2 * CS * K,), jnp.int32),
            pltpu.VMEM((2 * CS * K, F), jnp.float32),
            pltpu.VMEM((2 * CS * K, F), jnp.float32),
            pltpu.SemaphoreType.DMA,
            pltpu.SemaphoreType.DMA,
            pltpu.SemaphoreType.DMA,
            pltpu.SemaphoreType.DMA,
        ],
    )


def kernel(receivers, senders):
    sf = senders.reshape(NSTOT, F)
    rf = receivers.reshape(B * NR, F)
    table = jnp.concatenate([sf, rf], axis=0)
    adj, idxg = _topk_call(senders, receivers)
    halves = _sc_gather_fn()(table, idxg.reshape(NE))
    edges = halves.reshape(NE, 2 * F)
    return edges, adj


# speedup vs baseline: 1.3013x; 1.3013x over previous
import functools

import jax
import jax.numpy as jnp
from jax import lax
from jax.experimental import pallas as pl
from jax.experimental.pallas import tpu as pltpu
from jax.experimental.pallas import tpu_sc as plsc

B, NS, NR, F, K = 4, 2048, 2048, 128, 16
NEB = NS * K

BS = 256

NW = 32
SPW = NS // NW
CS = 4
CHUNKS = SPW // CS


def _topk_body(s_ref, r_ref, adj_ref, idx_ref):
    s = s_ref[...]
    r = r_ref[...]
    s2 = jnp.sum(s * s, axis=1, keepdims=True)
    r2 = jnp.sum(r * r, axis=1)[None, :]
    mixed = lax.dot_general(s, r, (((1,), (1,)), ((), ())),
                            preferred_element_type=jnp.float32)
    d = jnp.abs(s2 + r2 - 2.0 * mixed)
    col = lax.broadcasted_iota(jnp.int32, (BS, NR), 1)
    kcol = lax.broadcasted_iota(jnp.int32, (BS, K), 1)
    idxs = jnp.zeros((BS, K), jnp.int32)
    inf = jnp.float32(jnp.inf)
    for k in range(K):
        m = jnp.min(d, axis=1, keepdims=True)
        sel = jnp.min(jnp.where(d <= m, col, NR), axis=1, keepdims=True)
        d = jnp.where(col == sel, inf, d)
        idxs = jnp.where(kcol == k, sel + NS, idxs)
    adj_ref[...] = (d == inf).astype(jnp.float32)
    idx_ref[...] = idxs


def _topk_call(senders_b, receivers_b):
    return pl.pallas_call(
        _topk_body,
        grid=(NS // BS,),
        in_specs=[
            pl.BlockSpec((BS, F), lambda i: (i, 0)),
            pl.BlockSpec((NR, F), lambda i: (0, 0)),
        ],
        out_specs=[
            pl.BlockSpec((BS, NR), lambda i: (i, 0)),
            pl.BlockSpec((BS, K), lambda i: (i, 0)),
        ],
        out_shape=[
            jax.ShapeDtypeStruct((NS, NR), jnp.float32),
            jax.ShapeDtypeStruct((NS, K), jnp.int32),
        ],
    )(senders_b, receivers_b)


def _sc_gather_body(table_hbm, idxg_hbm, out_hbm, idx_all,
                    gidx0, gidx1, gv0, gv1, gs0, gs1, ws0, ws1):
    wid = lax.axis_index("s") * 2 + lax.axis_index("c")
    sender_base = wid * SPW
    pltpu.sync_copy(idxg_hbm.at[pl.ds(sender_base * K, SPW * K)], idx_all)
    lane = lax.iota(jnp.int32, 16)
    gidx = (gidx0, gidx1)
    gv = (gv0, gv1)
    gs = (gs0, gs1)
    ws = (ws0, ws1)
    HR = 2 * CS * K

    def fill_and_fire(c, b):
        for i in range(CS):
            off = c * (CS * K) + i * K
            v = idx_all[pl.ds(off, K)]
            sk, _ = plsc.sort_key_val(v, v)
            sg = sender_base + c * CS + i
            splat = jnp.broadcast_to(sg, (16,)).astype(jnp.int32)
            pos_e = 2 * lane + (2 * K) * i
            plsc.store_scatter(gidx[b], [pos_e], splat)
            plsc.store_scatter(gidx[b], [pos_e + 1], sk)
        pltpu.make_async_copy(table_hbm.at[gidx[b]], gv[b], gs[b]).start()

    def out_slice(c):
        return out_hbm.at[pl.ds(sender_base * (2 * K) + c * HR, HR)]

    def drain_write(c, b):
        pltpu.make_async_copy(gv[b], out_slice(c), ws[b]).wait()

    def write_start(c, b):
        pltpu.make_async_copy(gv[b], out_slice(c), ws[b]).start()

    def wait_gather(b):
        pltpu.make_async_copy(table_hbm.at[gidx[b]], gv[b], gs[b]).wait()

    fill_and_fire(0, 0)

    def body(g, carry):
        c1 = 2 * g + 1

        @pl.when(g > 0)
        def _():
            drain_write(c1 - 2, 1)
        fill_and_fire(c1, 1)
        wait_gather(0)
        write_start(2 * g, 0)
        c2 = 2 * g + 2

        @pl.when(c2 < CHUNKS)
        def _():
            drain_write(c2 - 2, 0)
            fill_and_fire(c2, 0)
        wait_gather(1)
        write_start(c1, 1)
        return carry

    lax.fori_loop(0, CHUNKS // 2, body, 0)
    drain_write(CHUNKS - 2, 0)
    drain_write(CHUNKS - 1, 1)


@functools.cache
def _sc_gather_fn():
    mesh = plsc.VectorSubcoreMesh(core_axis_name="c", subcore_axis_name="s")
    return pl.kernel(
        _sc_gather_body,
        mesh=mesh,
        compiler_params=pltpu.CompilerParams(needs_layout_passes=False),
        out_type=jax.ShapeDtypeStruct((2 * NEB, F), jnp.float32),
        scratch_types=[
            pltpu.VMEM((SPW * K,), jnp.int32),
            pltpu.VMEM((2 * CS * K,), jnp.int32),
            pltpu.VMEM((2 * CS * K,), jnp.int32),
            pltpu.VMEM((2 * CS * K, F), jnp.float32),
            pltpu.VMEM((2 * CS * K, F), jnp.float32),
            pltpu.SemaphoreType.DMA,
            pltpu.SemaphoreType.DMA,
            pltpu.SemaphoreType.DMA,
            pltpu.SemaphoreType.DMA,
        ],
    )


def kernel(receivers, senders):
    sc = _sc_gather_fn()
    adjs, halves = [], []
    for b in range(B):
        table = jnp.concatenate([senders[b], receivers[b]], axis=0)
        adj_b, idx_b = _topk_call(senders[b], receivers[b])
        adjs.append(adj_b)
        halves.append(sc(table, idx_b.reshape(NEB)))
    adj = jnp.stack(adjs, axis=0)
    edges = jnp.concatenate(halves, axis=0).reshape(B * NEB, 2 * F)
    return edges, adj


# speedup vs baseline: 1.4408x; 1.1072x over previous
import functools

import jax
import jax.numpy as jnp
from jax import lax
from jax.experimental import pallas as pl
from jax.experimental.pallas import tpu as pltpu
from jax.experimental.pallas import tpu_sc as plsc

B, NS, NR, F, K = 4, 2048, 2048, 128, 16
NSTOT = B * NS
NE = NSTOT * K

BS = 256

NW = 32
SPW = NSTOT // NW
CS = 4
CHUNKS = SPW // CS
NBUF = 4


def _topk_body(s_ref, r_ref, adj_ref, idx_ref):
    b = pl.program_id(0)
    s = s_ref[0]
    r = r_ref[0]
    s2 = jnp.sum(s * s, axis=1, keepdims=True)
    r2 = jnp.sum(r * r, axis=1)[None, :]
    mixed = lax.dot_general(s, r, (((1,), (1,)), ((), ())),
                            preferred_element_type=jnp.float32)
    d = jnp.abs(s2 + r2 - 2.0 * mixed)
    col = lax.broadcasted_iota(jnp.int32, (BS, NR), 1)
    kcol = lax.broadcasted_iota(jnp.int32, (BS, K), 1)
    offset = NSTOT + b * NR
    idxs = jnp.zeros((BS, K), jnp.int32)
    inf = jnp.float32(jnp.inf)
    for k in range(K):
        m = jnp.min(d, axis=1, keepdims=True)
        sel = jnp.min(jnp.where(d <= m, col, NR), axis=1, keepdims=True)
        d = jnp.where(col == sel, inf, d)
        idxs = jnp.where(kcol == k, sel + offset, idxs)
    adj_ref[0] = (d == inf).astype(jnp.float32)
    idx_ref[0] = idxs


def _topk_call(senders, receivers):
    return pl.pallas_call(
        _topk_body,
        grid=(B, NS // BS),
        in_specs=[
            pl.BlockSpec((1, BS, F), lambda b, i: (b, i, 0)),
            pl.BlockSpec((1, NR, F), lambda b, i: (b, 0, 0)),
        ],
        out_specs=[
            pl.BlockSpec((1, BS, NR), lambda b, i: (b, i, 0)),
            pl.BlockSpec((1, BS, K), lambda b, i: (b, i, 0)),
        ],
        out_shape=[
            jax.ShapeDtypeStruct((B, NS, NR), jnp.float32),
            jax.ShapeDtypeStruct((B, NS, K), jnp.int32),
        ],
    )(senders, receivers)


def _sc_gather_body(table_hbm, idxg_hbm, out_hbm, idx_all,
                    gidx0, gidx1, gidx2, gidx3,
                    gv0, gv1, gv2, gv3,
                    gs0, gs1, gs2, gs3, ws0, ws1, ws2, ws3):
    wid = lax.axis_index("s") * 2 + lax.axis_index("c")
    sender_base = wid * SPW
    pltpu.sync_copy(idxg_hbm.at[pl.ds(sender_base * K, SPW * K)], idx_all)
    lane = lax.iota(jnp.int32, 16)
    gidx = (gidx0, gidx1, gidx2, gidx3)
    gv = (gv0, gv1, gv2, gv3)
    gs = (gs0, gs1, gs2, gs3)
    ws = (ws0, ws1, ws2, ws3)
    HR = 2 * CS * K

    def fill_and_fire(c, b):
        for i in range(CS):
            off = c * (CS * K) + i * K
            v = idx_all[pl.ds(off, K)]
            sk, _ = plsc.sort_key_val(v, v)
            sg = sender_base + c * CS + i
            splat = jnp.broadcast_to(sg, (16,)).astype(jnp.int32)
            pos_e = 2 * lane + (2 * K) * i
            plsc.store_scatter(gidx[b], [pos_e], splat)
            plsc.store_scatter(gidx[b], [pos_e + 1], sk)
        pltpu.make_async_copy(table_hbm.at[gidx[b]], gv[b], gs[b]).start()

    def out_slice(c):
        return out_hbm.at[pl.ds(sender_base * (2 * K) + c * HR, HR)]

    def drain_write(c, b):
        pltpu.make_async_copy(gv[b], out_slice(c), ws[b]).wait()

    def write_start(c, b):
        pltpu.make_async_copy(gv[b], out_slice(c), ws[b]).start()

    def wait_gather(b):
        pltpu.make_async_copy(table_hbm.at[gidx[b]], gv[b], gs[b]).wait()

    for c0 in range(NBUF - 1):
        fill_and_fire(c0, c0)

    def body(g, carry):
        for b in range(NBUF):
            c = NBUF * g + b
            wait_gather(b)
            write_start(c, b)
            n = c + (NBUF - 1)
            bn = (b + NBUF - 1) % NBUF
            fire_ok = n < CHUNKS
            drain_ok = jnp.logical_and(fire_ok, n - NBUF >= 0)

            @pl.when(drain_ok)
            def _():
                drain_write(n - NBUF, bn)

            @pl.when(fire_ok)
            def _():
                fill_and_fire(n, bn)
        return carry

    lax.fori_loop(0, CHUNKS // NBUF, body, 0)
    for c in range(CHUNKS - NBUF, CHUNKS):
        drain_write(c, c % NBUF)


@functools.cache
def _sc_gather_fn():
    mesh = plsc.VectorSubcoreMesh(core_axis_name="c", subcore_axis_name="s")
    return pl.kernel(
        _sc_gather_body,
        mesh=mesh,
        compiler_params=pltpu.CompilerParams(needs_layout_passes=False),
        out_type=jax.ShapeDtypeStruct((2 * NE, F), jnp.float32),
        scratch_types=(
            [pltpu.VMEM((SPW * K,), jnp.int32)]
            + [pltpu.VMEM((2 * CS * K,), jnp.int32) for _ in range(NBUF)]
            + [pltpu.VMEM((2 * CS * K, F), jnp.float32) for _ in range(NBUF)]
            + [pltpu.SemaphoreType.DMA for _ in range(2 * NBUF)]
        ),
    )


def kernel(receivers, senders):
    sf = senders.reshape(NSTOT, F)
    rf = receivers.reshape(B * NR, F)
    table = jnp.concatenate([sf, rf], axis=0)
    adj, idxg = _topk_call(senders, receivers)
    halves = _sc_gather_fn()(table, idxg.reshape(NE))
    edges = halves.reshape(NE, 2 * F)
    return edges, adj


# speedup vs baseline: 1.4419x; 1.0008x over previous
import functools

import jax
import jax.numpy as jnp
from jax import lax
from jax.experimental import pallas as pl
from jax.experimental.pallas import tpu as pltpu
from jax.experimental.pallas import tpu_sc as plsc

B, NS, NR, F, K = 4, 2048, 2048, 128, 16
NSTOT = B * NS
NE = NSTOT * K

BS = 256

NW = 32
SPW = NSTOT // NW
CS = 8
CHUNKS = SPW // CS
NBUF = 2
HU = 128


def _topk_body(s_ref, r_ref, adj_ref, idx_ref):
    b = pl.program_id(0)
    s = s_ref[0]
    r = r_ref[0]
    s2 = jnp.sum(s * s, axis=1, keepdims=True)
    r2 = jnp.sum(r * r, axis=1)[None, :]
    mixed = lax.dot_general(s, r, (((1,), (1,)), ((), ())),
                            preferred_element_type=jnp.float32)
    d = jnp.abs(s2 + r2 - 2.0 * mixed)
    col = lax.broadcasted_iota(jnp.int32, (BS, NR), 1)
    kcol = lax.broadcasted_iota(jnp.int32, (BS, K), 1)
    offset = NSTOT + b * NR
    idxs = jnp.zeros((BS, K), jnp.int32)
    inf = jnp.float32(jnp.inf)
    for k in range(K):
        m = jnp.min(d, axis=1, keepdims=True)
        sel = jnp.min(jnp.where(d <= m, col, NR), axis=1, keepdims=True)
        d = jnp.where(col == sel, inf, d)
        idxs = jnp.where(kcol == k, sel + offset, idxs)
    adj_ref[0] = (d == inf).astype(jnp.float32)
    idx_ref[0] = idxs


def _topk_call(senders, receivers):
    return pl.pallas_call(
        _topk_body,
        grid=(B, NS // BS),
        in_specs=[
            pl.BlockSpec((1, BS, F), lambda b, i: (b, i, 0)),
            pl.BlockSpec((1, NR, F), lambda b, i: (b, 0, 0)),
        ],
        out_specs=[
            pl.BlockSpec((1, BS, NR), lambda b, i: (b, i, 0)),
            pl.BlockSpec((1, BS, K), lambda b, i: (b, i, 0)),
        ],
        out_shape=[
            jax.ShapeDtypeStruct((B, NS, NR), jnp.float32),
            jax.ShapeDtypeStruct((B, NS, K), jnp.int32),
        ],
    )(senders, receivers)


def _sc_gather_body(table_hbm, idxg_hbm, out_hbm, idx_all,
                    gidx0a, gidx0b, gidx1a, gidx1b,
                    gv0, gv1, gs0, gs1, ws0, ws1):
    wid = lax.axis_index("s") * 2 + lax.axis_index("c")
    sender_base = wid * SPW
    base_u = sender_base * (2 * K) // HU
    pltpu.sync_copy(idxg_hbm.at[pl.ds(sender_base * K, SPW * K)], idx_all)
    lane = lax.iota(jnp.int32, 16)
    gidx = ((gidx0a, gidx0b), (gidx1a, gidx1b))
    gv = (gv0, gv1)
    gs = (gs0, gs1)
    ws = (ws0, ws1)
    UPC = 2 * CS * K // HU

    def fill_and_fire(c, b):
        for i in range(CS):
            off = c * (CS * K) + i * K
            v = idx_all[pl.ds(off, K)]
            sk, _ = plsc.sort_key_val(v, v)
            sg = sender_base + c * CS + i
            splat = jnp.broadcast_to(sg, (16,)).astype(jnp.int32)
            half = gidx[b][i // 4]
            pos_e = 2 * lane + (2 * K) * (i % 4)
            plsc.store_scatter(half, [pos_e], splat)
            plsc.store_scatter(half, [pos_e + 1], sk)
        pltpu.make_async_copy(table_hbm.at[gidx[b][0]], gv[b].at[0], gs[b]).start()
        pltpu.make_async_copy(table_hbm.at[gidx[b][1]], gv[b].at[1], gs[b]).start()

    def out_slice(c):
        return out_hbm.at[pl.ds(base_u + c * UPC, UPC)]

    def drain_write(c, b):
        pltpu.make_async_copy(gv[b], out_slice(c), ws[b]).wait()

    def write_start(c, b):
        pltpu.make_async_copy(gv[b], out_slice(c), ws[b]).start()

    def wait_gather(b):
        pltpu.make_async_copy(table_hbm.at[gidx[b][0]], gv[b].at[0], gs[b]).wait()
        pltpu.make_async_copy(table_hbm.at[gidx[b][1]], gv[b].at[1], gs[b]).wait()

    fill_and_fire(0, 0)

    def body(g, carry):
        c1 = 2 * g + 1

        @pl.when(g > 0)
        def _():
            drain_write(c1 - 2, 1)
        fill_and_fire(c1, 1)
        wait_gather(0)
        write_start(2 * g, 0)
        c2 = 2 * g + 2

        @pl.when(c2 < CHUNKS)
        def _():
            drain_write(c2 - 2, 0)
            fill_and_fire(c2, 0)
        wait_gather(1)
        write_start(c1, 1)
        return carry

    lax.fori_loop(0, CHUNKS // 2, body, 0)
    drain_write(CHUNKS - 2, 0)
    drain_write(CHUNKS - 1, 1)


@functools.cache
def _sc_gather_fn():
    mesh = plsc.VectorSubcoreMesh(core_axis_name="c", subcore_axis_name="s")
    return pl.kernel(
        _sc_gather_body,
        mesh=mesh,
        compiler_params=pltpu.CompilerParams(needs_layout_passes=False),
        out_type=jax.ShapeDtypeStruct((2 * NE // HU, HU, F), jnp.float32),
        scratch_types=(
            [pltpu.VMEM((SPW * K,), jnp.int32)]
            + [pltpu.VMEM((HU,), jnp.int32) for _ in range(2 * NBUF)]
            + [pltpu.VMEM((2 * CS * K // HU, HU, F), jnp.float32)
               for _ in range(NBUF)]
            + [pltpu.SemaphoreType.DMA for _ in range(2 * NBUF)]
        ),
    )


def kernel(receivers, senders):
    sf = senders.reshape(NSTOT, F)
    rf = receivers.reshape(B * NR, F)
    table = jnp.concatenate([sf, rf], axis=0)
    adj, idxg = _topk_call(senders, receivers)
    halves = _sc_gather_fn()(table, idxg.reshape(NE))
    edges = halves.reshape(NE, 2 * F)
    return edges, adj


# speedup vs baseline: 1.5923x; 1.1043x over previous
import functools

import jax
import jax.numpy as jnp
from jax import lax
from jax.experimental import pallas as pl
from jax.experimental.pallas import tpu as pltpu
from jax.experimental.pallas import tpu_sc as plsc

B, NS, NR, F, K = 4, 2048, 2048, 128, 16
NSTOT = B * NS
NE = NSTOT * K

BS = 256

NW = 32
SPW = NSTOT // NW
CS = 8
CHUNKS = SPW // CS
NBUF = 2
HU = 128


def _topk_body(s_ref, r_ref, adj_ref, idx_ref):
    b = pl.program_id(0)
    s = s_ref[0]
    r = r_ref[0]
    s2 = jnp.sum(s * s, axis=1, keepdims=True)
    r2 = jnp.sum(r * r, axis=1)[None, :]
    mixed = lax.dot_general(s, r, (((1,), (1,)), ((), ())),
                            preferred_element_type=jnp.float32)
    d = jnp.abs(s2 + r2 - 2.0 * mixed)
    col = lax.broadcasted_iota(jnp.int32, (BS, NR), 1)
    kcol = lax.broadcasted_iota(jnp.int32, (BS, K), 1)
    offset = NSTOT + b * NR
    idxs = jnp.zeros((BS, K), jnp.int32)
    inf = jnp.float32(jnp.inf)
    for k in range(K):
        m = jnp.min(d, axis=1, keepdims=True)
        sel = jnp.min(jnp.where(d <= m, col, NR), axis=1, keepdims=True)
        d = jnp.where(col == sel, inf, d)
        idxs = jnp.where(kcol == k, sel + offset, idxs)
    adj_ref[0] = (d == inf).astype(jnp.float32)
    idx_ref[0] = idxs


def _topk_call(senders, receivers):
    return pl.pallas_call(
        _topk_body,
        grid=(B, NS // BS),
        in_specs=[
            pl.BlockSpec((1, BS, F), lambda b, i: (b, i, 0)),
            pl.BlockSpec((1, NR, F), lambda b, i: (b, 0, 0)),
        ],
        out_specs=[
            pl.BlockSpec((1, BS, NR), lambda b, i: (b, i, 0)),
            pl.BlockSpec((1, BS, K), lambda b, i: (b, i, 0)),
        ],
        out_shape=[
            jax.ShapeDtypeStruct((B, NS, NR), jnp.float32),
            jax.ShapeDtypeStruct((B, NS, K), jnp.int32),
        ],
    )(senders, receivers)


def _sc_gather_body(table_hbm, tflat_hbm, idxg_hbm, out_hbm, idx_all, sall,
                    gidx0, gidx1, lv0, lv1, rv0, rv1, gs0, gs1, ws0, ws1):
    wid = lax.axis_index("s") * 2 + lax.axis_index("c")
    sender_base = wid * SPW
    pltpu.sync_copy(idxg_hbm.at[pl.ds(sender_base * K, SPW * K)], idx_all)
    pltpu.sync_copy(tflat_hbm.at[pl.ds(sender_base * F, SPW * F)], sall)
    lane = lax.iota(jnp.int32, 16)
    gidx = (gidx0, gidx1)
    lv = (lv0, lv1)
    rv = (rv0, rv1)
    gs = (gs0, gs1)
    ws = (ws0, ws1)

    def fill_and_fire(c, b):
        for i in range(CS):
            off = c * (CS * K) + i * K
            v = idx_all[pl.ds(off, K)]
            sk, _ = plsc.sort_key_val(v, v)
            plsc.store_scatter(gidx[b], [lane + K * i], sk)
        pltpu.make_async_copy(table_hbm.at[gidx[b]], rv[b], gs[b]).start()
        for i in range(CS):
            srow = (c * CS + i) * F
            for vv in range(F // 16):
                vec = sall[pl.ds(srow + 16 * vv, 16)]
                for j in range(K):
                    lv[b][K * i + j, pl.ds(16 * vv, 16)] = vec

    def drain_write(c, b):
        row0 = sender_base * K + c * (CS * K)
        pltpu.make_async_copy(lv[b], out_hbm.at[pl.ds(row0, HU), 0], ws[b]).wait()
        pltpu.make_async_copy(rv[b], out_hbm.at[pl.ds(row0, HU), 1], ws[b]).wait()

    def write_start(c, b):
        row0 = sender_base * K + c * (CS * K)
        pltpu.make_async_copy(lv[b], out_hbm.at[pl.ds(row0, HU), 0], ws[b]).start()
        pltpu.make_async_copy(rv[b], out_hbm.at[pl.ds(row0, HU), 1], ws[b]).start()

    def wait_gather(b):
        pltpu.make_async_copy(table_hbm.at[gidx[b]], rv[b], gs[b]).wait()

    fill_and_fire(0, 0)

    def body(g, carry):
        c1 = 2 * g + 1

        @pl.when(g > 0)
        def _():
            drain_write(c1 - 2, 1)
        fill_and_fire(c1, 1)
        wait_gather(0)
        write_start(2 * g, 0)
        c2 = 2 * g + 2

        @pl.when(c2 < CHUNKS)
        def _():
            drain_write(c2 - 2, 0)
            fill_and_fire(c2, 0)
        wait_gather(1)
        write_start(c1, 1)
        return carry

    lax.fori_loop(0, CHUNKS // 2, body, 0)
    drain_write(CHUNKS - 2, 0)
    drain_write(CHUNKS - 1, 1)


@functools.cache
def _sc_gather_fn():
    mesh = plsc.VectorSubcoreMesh(core_axis_name="c", subcore_axis_name="s")
    return pl.kernel(
        _sc_gather_body,
        mesh=mesh,
        compiler_params=pltpu.CompilerParams(needs_layout_passes=False),
        out_type=jax.ShapeDtypeStruct((NE, 2, F), jnp.float32),
        scratch_types=(
            [pltpu.VMEM((SPW * K,), jnp.int32),
             pltpu.VMEM((SPW * F,), jnp.float32)]
            + [pltpu.VMEM((HU,), jnp.int32) for _ in range(NBUF)]
            + [pltpu.VMEM((HU, F), jnp.float32) for _ in range(2 * NBUF)]
            + [pltpu.SemaphoreType.DMA for _ in range(2 * NBUF)]
        ),
    )


def kernel(receivers, senders):
    sf = senders.reshape(NSTOT, F)
    rf = receivers.reshape(B * NR, F)
    table = jnp.concatenate([sf, rf], axis=0)
    adj, idxg = _topk_call(senders, receivers)
    halves = _sc_gather_fn()(table, table.reshape(-1), idxg.reshape(NE))
    edges = halves.reshape(NE, 2 * F)
    return edges, adj
